# baseline (speedup 1.0000x reference)
import functools

import jax
import jax.numpy as jnp
from jax.experimental import pallas as pl
from jax.experimental.pallas import tpu as pltpu

_PLANES = [32, 64, 128, 256, 512]
_STRIDE = [1, 4, 4, 4, 4]
_NSAMPLE = [8, 16, 16, 16, 16]
_SHARE = 8


def _rup(v, m):
    return -(-v // m) * m


def _fps_body(sx_ref, sy_ref, sz_ref, px_ref, py_ref, pz_ref, oidx_ref,
              *, m, n):
    px = px_ref[...]
    py = py_ref[...]
    pz = pz_ref[...]
    rows = px.shape[0]
    lin = (jax.lax.broadcasted_iota(jnp.int32, (rows, 128), 0) * 128
           + jax.lax.broadcasted_iota(jnp.int32, (rows, 128), 1))
    mrows = oidx_ref.shape[0]
    mlin = (jax.lax.broadcasted_iota(jnp.int32, (mrows, 128), 0) * 128
            + jax.lax.broadcasted_iota(jnp.int32, (mrows, 128), 1))
    dmin0 = jnp.where(lin < n, jnp.float32(1e10), jnp.float32(-jnp.inf))
    idxs0 = jnp.zeros((mrows, 128), jnp.int32)

    def body(i, carry):
        last, dmin, idxs = carry
        lx = sx_ref[last]
        ly = sy_ref[last]
        lz = sz_ref[last]
        dx = px - lx
        dy = py - ly
        dz = pz - lz
        d = dx * dx + dy * dy + dz * dz
        dmin = jnp.minimum(dmin, d)
        mx = jnp.max(dmin)
        nidx = jnp.min(jnp.where(dmin == mx, lin, jnp.int32(2147483647)))
        idxs = jnp.where(mlin == i, nidx, idxs)
        return nidx, dmin, idxs

    _, _, idxs = jax.lax.fori_loop(1, m, body, (jnp.int32(0), dmin0, idxs0))
    oidx_ref[...] = idxs


def _fps(p, m):
    n = p.shape[0]
    npad = _rup(n, 128)
    rows = npad // 128
    mpad = _rup(m, 128)
    mrows = mpad // 128
    pp = jnp.pad(p, ((0, npad - n), (0, 0)))
    px = pp[:, 0].reshape(rows, 128)
    py = pp[:, 1].reshape(rows, 128)
    pz = pp[:, 2].reshape(rows, 128)
    smem = pl.BlockSpec(memory_space=pltpu.SMEM)
    vmem = pl.BlockSpec(memory_space=pltpu.VMEM)
    out = pl.pallas_call(
        functools.partial(_fps_body, m=m, n=n),
        in_specs=[smem, smem, smem, vmem, vmem, vmem],
        out_shape=jax.ShapeDtypeStruct((mrows, 128), jnp.int32),
    )(pp[:, 0], pp[:, 1], pp[:, 2], px, py, pz)
    return out.reshape(-1)[:m]


def _knn_body(qx_ref, qy_ref, qz_ref, rx_ref, ry_ref, rz_ref, oi_ref, od_ref,
              *, k, nr):
    dx = qx_ref[...] - rx_ref[...]
    dy = qy_ref[...] - ry_ref[...]
    dz = qz_ref[...] - rz_ref[...]
    dist = dx * dx + dy * dy + dz * dz
    t, nrp = dist.shape
    col = jax.lax.broadcasted_iota(jnp.int32, (t, nrp), 1)
    dist = jnp.where(col < nr, dist, jnp.float32(jnp.inf))
    ocol = jax.lax.broadcasted_iota(jnp.int32, (t, 128), 1)
    oi = jnp.zeros((t, 128), jnp.int32)
    od = jnp.zeros((t, 128), jnp.float32)
    for j in range(k):
        mval = jnp.min(dist, axis=1, keepdims=True)
        sel = jnp.min(
            jnp.where(dist == mval, col, jnp.int32(2147483647)),
            axis=1, keepdims=True)
        oi = jnp.where(ocol == j, sel, oi)
        od = jnp.where(ocol == j, mval, od)
        dist = jnp.where(col == sel, jnp.float32(jnp.inf), dist)
    oi_ref[...] = oi
    od_ref[...] = od


def _knn(q, r, k):
    nq, nr = q.shape[0], r.shape[0]
    nq_pad = _rup(nq, 8)
    t = min(512, nq_pad)
    nq_pad = _rup(nq_pad, t)
    nr_pad = _rup(nr, 128)
    qp = jnp.pad(q, ((0, nq_pad - nq), (0, 0)))
    rp = jnp.pad(r, ((0, nr_pad - nr), (0, 0)))
    qx, qy, qz = qp[:, 0:1], qp[:, 1:2], qp[:, 2:3]
    rx = rp[:, 0].reshape(1, nr_pad)
    ry = rp[:, 1].reshape(1, nr_pad)
    rz = rp[:, 2].reshape(1, nr_pad)
    grid = (nq_pad // t,)
    qspec = pl.BlockSpec((t, 1), lambda i: (i, 0))
    rspec = pl.BlockSpec((1, nr_pad), lambda i: (0, 0))
    ospec = pl.BlockSpec((t, 128), lambda i: (i, 0))
    oi, od = pl.pallas_call(
        functools.partial(_knn_body, k=k, nr=nr),
        grid=grid,
        in_specs=[qspec, qspec, qspec, rspec, rspec, rspec],
        out_specs=[ospec, ospec],
        out_shape=[
            jax.ShapeDtypeStruct((nq_pad, 128), jnp.int32),
            jax.ShapeDtypeStruct((nq_pad, 128), jnp.float32),
        ],
    )(qx, qy, qz, rx, ry, rz)
    return oi[:nq, :k], od[:nq, :k]


def _blk_body(*refs, n, k):
    (x_ref, pr_ref, idxf_ref,
     l1w, bn1g, bn1b, qw, qb, kw, kb, vw, vb,
     p1w, p1b, pbng, pbnb, p2w, p2b,
     wbn1g, wbn1b, w1w, w1b, wbn2g, wbn2b, w2w, w2b,
     bn2g, bn2b, l3w, bn3g, bn3b, o_ref) = refs
    f32 = jnp.float32
    x = x_ref[...]
    c = x.shape[1]
    s = c // _SHARE
    nk = n * k

    def mm(a, b):
        return jnp.dot(a, b, preferred_element_type=f32,
                       precision=jax.lax.Precision.HIGHEST)

    def sel_mm(a, b):
        return jnp.dot(a, b, preferred_element_type=f32,
                       precision=jax.lax.Precision.HIGHEST)

    def bn0(v, g_ref, b_ref):
        mu = jnp.mean(v, axis=0, keepdims=True)
        dv = v - mu
        va = jnp.mean(dv * dv, axis=0, keepdims=True)
        return dv / jnp.sqrt(va + 1e-5) * g_ref[...] + b_ref[...]

    y = jnp.maximum(bn0(mm(x, l1w[...]), bn1g, bn1b), 0.0)
    xq = mm(y, qw[...]) + qb[...]
    xk = mm(y, kw[...]) + kb[...]
    xv = mm(y, vw[...]) + vb[...]
    cols = jax.lax.broadcasted_iota(jnp.int32, (nk, n), 1)
    gsel = (cols == idxf_ref[...]).astype(f32)
    rsel = (cols == jax.lax.broadcasted_iota(jnp.int32, (nk, n), 0)
            // k).astype(f32)
    xkg = sel_mm(gsel, xk)
    xvg = sel_mm(gsel, xv)
    xqr = sel_mm(rsel, xq)
    pe = mm(pr_ref[...], p1w[...]) + p1b[...]
    pe = jnp.maximum(bn0(pe, pbng, pbnb), 0.0)
    pe = mm(pe, p2w[...]) + p2b[...]
    w = xkg - xqr + pe
    w = jnp.maximum(bn0(w, wbn1g, wbn1b), 0.0)
    w = mm(w, w1w[...]) + w1b[...]
    w = jnp.maximum(bn0(w, wbn2g, wbn2b), 0.0)
    w = mm(w, w2w[...]) + w2b[...]
    e = jnp.exp(w - jnp.max(w))
    bsel = (jax.lax.broadcasted_iota(jnp.int32, (n, nk), 1) // k ==
            jax.lax.broadcasted_iota(jnp.int32, (n, nk), 0)).astype(f32)
    sm = e / sel_mm(rsel, sel_mm(bsel, e))
    smf = jnp.concatenate([sm] * _SHARE, axis=1)
    agg = sel_mm(bsel, (xvg + pe) * smf)
    y2 = jnp.maximum(bn0(agg, bn2g, bn2b), 0.0)
    y3 = bn0(mm(y2, l3w[...]), bn3g, bn3b)
    o_ref[...] = jnp.maximum(y3 + x, 0.0)


def _r2(a):
    return a.reshape(1, -1)


def _pt_block_fused(bp, x, prf, idxf, n, k):
    c = x.shape[1]
    tr = bp["tr"]
    args = [
        x, prf, idxf,
        bp["l1"]["w"], _r2(bp["bn1"]["g"]), _r2(bp["bn1"]["b"]),
        tr["q"]["w"], _r2(tr["q"]["b"]),
        tr["k"]["w"], _r2(tr["k"]["b"]),
        tr["v"]["w"], _r2(tr["v"]["b"]),
        tr["p1"]["w"], _r2(tr["p1"]["b"]),
        _r2(tr["pbn"]["g"]), _r2(tr["pbn"]["b"]),
        tr["p2"]["w"], _r2(tr["p2"]["b"]),
        _r2(tr["wbn1"]["g"]), _r2(tr["wbn1"]["b"]),
        tr["w1"]["w"], _r2(tr["w1"]["b"]),
        _r2(tr["wbn2"]["g"]), _r2(tr["wbn2"]["b"]),
        tr["w2"]["w"], _r2(tr["w2"]["b"]),
        _r2(bp["bn2"]["g"]), _r2(bp["bn2"]["b"]),
        bp["l3"]["w"], _r2(bp["bn3"]["g"]), _r2(bp["bn3"]["b"]),
    ]
    return pl.pallas_call(
        functools.partial(_blk_body, n=n, k=k),
        out_shape=jax.ShapeDtypeStruct((n, c), jnp.float32),
    )(*args)


def _relu(v):
    return jnp.maximum(v, 0.0)


def _lin(v, p):
    y = v @ p["w"]
    if "b" in p:
        y = y + p["b"]
    return y


def _bn(v, p, axes):
    m = jnp.mean(v, axis=axes, keepdims=True)
    var = jnp.var(v, axis=axes, keepdims=True)
    return (v - m) / jnp.sqrt(var + 1e-5) * p["g"] + p["b"]


def _pt_layer(pr, p, x, idx, nsample, share):
    n = x.shape[0]
    out = pr["q"]["w"].shape[1]
    xq = _lin(x, pr["q"])
    xk = _lin(x, pr["k"])
    xv = _lin(x, pr["v"])
    p_r = p[idx] - p[:, None, :]
    xk = xk[idx]
    xv = xv[idx]
    pe = _lin(p_r, pr["p1"])
    pe = _relu(_bn(pe, pr["pbn"], (0, 1)))
    pe = _lin(pe, pr["p2"])
    w = xk - xq[:, None, :] + pe
    w = _relu(_bn(w, pr["wbn1"], (0, 1)))
    w = _lin(w, pr["w1"])
    w = _relu(_bn(w, pr["wbn2"], (0, 1)))
    w = _lin(w, pr["w2"])
    w = jax.nn.softmax(w, axis=1)
    v = (xv + pe).reshape(n, nsample, share, out // share)
    return jnp.sum(v * w[:, :, None, :], axis=1).reshape(n, out)


def _pt_block(bp, p, x, idx, nsample, share):
    identity = x
    x = _relu(_bn(_lin(x, bp["l1"]), bp["bn1"], 0))
    x = _relu(_bn(_pt_layer(bp["tr"], p, x, idx, nsample, share),
                  bp["bn2"], 0))
    x = _bn(_lin(x, bp["l3"]), bp["bn3"], 0)
    return _relu(x + identity)


def _transition_down(tp, p, x, stride, nsample):
    if stride == 1:
        return p, _relu(_bn(_lin(x, tp["lin"]), tp["bn"], 0))
    m = p.shape[0] // stride
    sidx = _fps(p, m)
    n_p = p[sidx]
    nidx, _ = _knn(n_p, p, nsample)
    grouped = jnp.concatenate([p[nidx] - n_p[:, None, :], x[nidx]], axis=-1)
    y = _lin(grouped, tp["lin"])
    y = _relu(_bn(y, tp["bn"], (0, 1)))
    return n_p, jnp.max(y, axis=1)


def _tu_head(tp, x):
    g = jnp.mean(x, axis=0, keepdims=True)
    g = _relu(_lin(g, tp["l2"]))
    xc = jnp.concatenate([x, jnp.broadcast_to(g, x.shape)], axis=1)
    return _relu(_bn(_lin(xc, tp["l1"]), tp["l1bn"], 0))


def _tu(tp, p1, x1, p2, x2):
    a = _relu(_bn(_lin(x1, tp["l1"]), tp["l1bn"], 0))
    b = _relu(_bn(_lin(x2, tp["l2"]), tp["l2bn"], 0))
    idx, d2 = _knn(p1, p2, 3)
    w = 1.0 / (jnp.sqrt(jnp.maximum(d2, 1e-12)) + 1e-8)
    w = w / jnp.sum(w, axis=1, keepdims=True)
    return a + jnp.sum(b[idx] * w[:, :, None], axis=1)


def _run_blocks(blocks, p, feats, aux, ns):
    idx, idxf, prf = aux
    n = p.shape[0]
    for bp in blocks:
        if n <= 256:
            feats = _pt_block_fused(bp, feats, prf, idxf, n, ns)
        else:
            feats = _pt_block(bp, p, feats, idx, ns, _SHARE)
    return feats


def _forward(pos, x, params):
    feats = jnp.concatenate([pos, x], axis=1)
    p = pos
    skips = []
    self_aux = []
    for li in range(5):
        ep = params["enc%d" % (li + 1)]
        p, feats = _transition_down(ep["td"], p, feats, _STRIDE[li],
                                    _NSAMPLE[li])
        idx, _ = _knn(p, p, _NSAMPLE[li])
        idxf = idx.reshape(-1, 1)
        prf = (p[idx] - p[:, None, :]).reshape(-1, 3)
        aux = (idx, idxf, prf)
        self_aux.append(aux)
        feats = _run_blocks(ep["blocks"], p, feats, aux, _NSAMPLE[li])
        skips.append((p, feats))
    p5, x5 = skips[4]
    x5 = _tu_head(params["dec5"]["tu"], x5)
    x5 = _run_blocks(params["dec5"]["blocks"], p5, x5, self_aux[4],
                     _NSAMPLE[4])
    cur_p, cur_x = p5, x5
    for name, lv, ns in zip(["dec4", "dec3", "dec2", "dec1"], [3, 2, 1, 0],
                            [_NSAMPLE[3], _NSAMPLE[2], _NSAMPLE[1],
                             _NSAMPLE[0]]):
        p_l, x_l = skips[lv]
        x_new = _tu(params[name]["tu"], p_l, x_l, cur_p, cur_x)
        x_new = _run_blocks(params[name]["blocks"], p_l, x_new,
                            self_aux[lv], ns)
        cur_p, cur_x = p_l, x_new
    out = _lin(cur_x, params["cls"]["l1"])
    out = _relu(_bn(out, params["cls"]["bn"], 0))
    return _lin(out, params["cls"]["l2"])


@jax.jit
def _forward_jit(pos, x, params):
    return _forward(pos, x, params)


def kernel(pos, x, batch, params):
    return _forward_jit(pos, x, params)


# speedup vs baseline: 1.0098x; 1.0098x over previous
import functools

import jax
import jax.numpy as jnp
from jax.experimental import pallas as pl
from jax.experimental.pallas import tpu as pltpu

_PLANES = [32, 64, 128, 256, 512]
_STRIDE = [1, 4, 4, 4, 4]
_NSAMPLE = [8, 16, 16, 16, 16]
_SHARE = 8


def _rup(v, m):
    return -(-v // m) * m


def _fps_body(sx_ref, sy_ref, sz_ref, px_ref, py_ref, pz_ref, oidx_ref,
              *, m, n):
    px = px_ref[...]
    py = py_ref[...]
    pz = pz_ref[...]
    rows = px.shape[0]
    lin = (jax.lax.broadcasted_iota(jnp.int32, (rows, 128), 0) * 128
           + jax.lax.broadcasted_iota(jnp.int32, (rows, 128), 1))
    mrows = oidx_ref.shape[0]
    mlin = (jax.lax.broadcasted_iota(jnp.int32, (mrows, 128), 0) * 128
            + jax.lax.broadcasted_iota(jnp.int32, (mrows, 128), 1))
    dmin0 = jnp.where(lin < n, jnp.float32(1e10), jnp.float32(-jnp.inf))
    idxs0 = jnp.zeros((mrows, 128), jnp.int32)

    def body(i, carry):
        last, dmin, idxs = carry
        lx = sx_ref[last]
        ly = sy_ref[last]
        lz = sz_ref[last]
        dx = px - lx
        dy = py - ly
        dz = pz - lz
        d = dx * dx + dy * dy + dz * dz
        dmin = jnp.minimum(dmin, d)
        mx = jnp.max(dmin)
        nidx = jnp.min(jnp.where(dmin == mx, lin, jnp.int32(2147483647)))
        idxs = jnp.where(mlin == i, nidx, idxs)
        return nidx, dmin, idxs

    _, _, idxs = jax.lax.fori_loop(1, m, body, (jnp.int32(0), dmin0, idxs0))
    oidx_ref[...] = idxs


def _fps(p, m):
    n = p.shape[0]
    npad = _rup(n, 128)
    rows = npad // 128
    mpad = _rup(m, 128)
    mrows = mpad // 128
    pp = jnp.pad(p, ((0, npad - n), (0, 0)))
    px = pp[:, 0].reshape(rows, 128)
    py = pp[:, 1].reshape(rows, 128)
    pz = pp[:, 2].reshape(rows, 128)
    smem = pl.BlockSpec(memory_space=pltpu.SMEM)
    vmem = pl.BlockSpec(memory_space=pltpu.VMEM)
    out = pl.pallas_call(
        functools.partial(_fps_body, m=m, n=n),
        in_specs=[smem, smem, smem, vmem, vmem, vmem],
        out_shape=jax.ShapeDtypeStruct((mrows, 128), jnp.int32),
    )(pp[:, 0], pp[:, 1], pp[:, 2], px, py, pz)
    return out.reshape(-1)[:m]


def _knn_body(qx_ref, qy_ref, qz_ref, rx_ref, ry_ref, rz_ref, oi_ref, od_ref,
              *, k, nr):
    dx = qx_ref[...] - rx_ref[...]
    dy = qy_ref[...] - ry_ref[...]
    dz = qz_ref[...] - rz_ref[...]
    dist = dx * dx + dy * dy + dz * dz
    t, nrp = dist.shape
    col = jax.lax.broadcasted_iota(jnp.int32, (t, nrp), 1)
    dist = jnp.where(col < nr, dist, jnp.float32(jnp.inf))
    ocol = jax.lax.broadcasted_iota(jnp.int32, (t, 128), 1)
    oi = jnp.zeros((t, 128), jnp.int32)
    od = jnp.zeros((t, 128), jnp.float32)
    for j in range(k):
        mval = jnp.min(dist, axis=1, keepdims=True)
        sel = jnp.min(
            jnp.where(dist == mval, col, jnp.int32(2147483647)),
            axis=1, keepdims=True)
        oi = jnp.where(ocol == j, sel, oi)
        od = jnp.where(ocol == j, mval, od)
        dist = jnp.where(col == sel, jnp.float32(jnp.inf), dist)
    oi_ref[...] = oi
    od_ref[...] = od


def _knn(q, r, k):
    nq, nr = q.shape[0], r.shape[0]
    nq_pad = _rup(nq, 8)
    t = min(256, nq_pad)
    nq_pad = _rup(nq_pad, t)
    nr_pad = _rup(nr, 128)
    qp = jnp.pad(q, ((0, nq_pad - nq), (0, 0)))
    rp = jnp.pad(r, ((0, nr_pad - nr), (0, 0)))
    qx, qy, qz = qp[:, 0:1], qp[:, 1:2], qp[:, 2:3]
    rx = rp[:, 0].reshape(1, nr_pad)
    ry = rp[:, 1].reshape(1, nr_pad)
    rz = rp[:, 2].reshape(1, nr_pad)
    grid = (nq_pad // t,)
    qspec = pl.BlockSpec((t, 1), lambda i: (i, 0))
    rspec = pl.BlockSpec((1, nr_pad), lambda i: (0, 0))
    ospec = pl.BlockSpec((t, 128), lambda i: (i, 0))
    oi, od = pl.pallas_call(
        functools.partial(_knn_body, k=k, nr=nr),
        grid=grid,
        in_specs=[qspec, qspec, qspec, rspec, rspec, rspec],
        out_specs=[ospec, ospec],
        out_shape=[
            jax.ShapeDtypeStruct((nq_pad, 128), jnp.int32),
            jax.ShapeDtypeStruct((nq_pad, 128), jnp.float32),
        ],
    )(qx, qy, qz, rx, ry, rz)
    return oi[:nq, :k], od[:nq, :k]


def _blk_body(*refs, n, k):
    (x_ref, pr_ref, idxf_ref,
     l1w, bn1g, bn1b, qw, qb, kw, kb, vw, vb,
     p1w, p1b, pbng, pbnb, p2w, p2b,
     wbn1g, wbn1b, w1w, w1b, wbn2g, wbn2b, w2w, w2b,
     bn2g, bn2b, l3w, bn3g, bn3b, o_ref) = refs
    f32 = jnp.float32
    x = x_ref[...]
    c = x.shape[1]
    s = c // _SHARE
    nk = n * k

    def mm(a, b):
        return jnp.dot(a, b, preferred_element_type=f32,
                       precision=jax.lax.Precision.HIGHEST)

    def sel_mm(a, b):
        return jnp.dot(a, b, preferred_element_type=f32,
                       precision=jax.lax.Precision.HIGHEST)

    def bn0(v, g_ref, b_ref):
        mu = jnp.mean(v, axis=0, keepdims=True)
        dv = v - mu
        va = jnp.mean(dv * dv, axis=0, keepdims=True)
        return dv / jnp.sqrt(va + 1e-5) * g_ref[...] + b_ref[...]

    y = jnp.maximum(bn0(mm(x, l1w[...]), bn1g, bn1b), 0.0)
    xq = mm(y, qw[...]) + qb[...]
    xk = mm(y, kw[...]) + kb[...]
    xv = mm(y, vw[...]) + vb[...]
    cols = jax.lax.broadcasted_iota(jnp.int32, (nk, n), 1)
    gsel = (cols == idxf_ref[...]).astype(f32)
    rsel = (cols == jax.lax.broadcasted_iota(jnp.int32, (nk, n), 0)
            // k).astype(f32)
    xkg = sel_mm(gsel, xk)
    xvg = sel_mm(gsel, xv)
    xqr = sel_mm(rsel, xq)
    pe = mm(pr_ref[...], p1w[...]) + p1b[...]
    pe = jnp.maximum(bn0(pe, pbng, pbnb), 0.0)
    pe = mm(pe, p2w[...]) + p2b[...]
    w = xkg - xqr + pe
    w = jnp.maximum(bn0(w, wbn1g, wbn1b), 0.0)
    w = mm(w, w1w[...]) + w1b[...]
    w = jnp.maximum(bn0(w, wbn2g, wbn2b), 0.0)
    w = mm(w, w2w[...]) + w2b[...]
    e = jnp.exp(w - jnp.max(w))
    bsel = (jax.lax.broadcasted_iota(jnp.int32, (n, nk), 1) // k ==
            jax.lax.broadcasted_iota(jnp.int32, (n, nk), 0)).astype(f32)
    sm = e / sel_mm(rsel, sel_mm(bsel, e))
    smf = jnp.concatenate([sm] * _SHARE, axis=1)
    agg = sel_mm(bsel, (xvg + pe) * smf)
    y2 = jnp.maximum(bn0(agg, bn2g, bn2b), 0.0)
    y3 = bn0(mm(y2, l3w[...]), bn3g, bn3b)
    o_ref[...] = jnp.maximum(y3 + x, 0.0)


def _r2(a):
    return a.reshape(1, -1)


def _pt_block_fused(bp, x, prf, idxf, n, k):
    c = x.shape[1]
    tr = bp["tr"]
    args = [
        x, prf, idxf,
        bp["l1"]["w"], _r2(bp["bn1"]["g"]), _r2(bp["bn1"]["b"]),
        tr["q"]["w"], _r2(tr["q"]["b"]),
        tr["k"]["w"], _r2(tr["k"]["b"]),
        tr["v"]["w"], _r2(tr["v"]["b"]),
        tr["p1"]["w"], _r2(tr["p1"]["b"]),
        _r2(tr["pbn"]["g"]), _r2(tr["pbn"]["b"]),
        tr["p2"]["w"], _r2(tr["p2"]["b"]),
        _r2(tr["wbn1"]["g"]), _r2(tr["wbn1"]["b"]),
        tr["w1"]["w"], _r2(tr["w1"]["b"]),
        _r2(tr["wbn2"]["g"]), _r2(tr["wbn2"]["b"]),
        tr["w2"]["w"], _r2(tr["w2"]["b"]),
        _r2(bp["bn2"]["g"]), _r2(bp["bn2"]["b"]),
        bp["l3"]["w"], _r2(bp["bn3"]["g"]), _r2(bp["bn3"]["b"]),
    ]
    return pl.pallas_call(
        functools.partial(_blk_body, n=n, k=k),
        out_shape=jax.ShapeDtypeStruct((n, c), jnp.float32),
    )(*args)


def _relu(v):
    return jnp.maximum(v, 0.0)


def _lin(v, p):
    y = v @ p["w"]
    if "b" in p:
        y = y + p["b"]
    return y


def _bn(v, p, axes):
    m = jnp.mean(v, axis=axes, keepdims=True)
    var = jnp.var(v, axis=axes, keepdims=True)
    return (v - m) / jnp.sqrt(var + 1e-5) * p["g"] + p["b"]


def _pt_layer(pr, p, x, idx, nsample, share):
    n = x.shape[0]
    out = pr["q"]["w"].shape[1]
    xq = _lin(x, pr["q"])
    xk = _lin(x, pr["k"])
    xv = _lin(x, pr["v"])
    p_r = p[idx] - p[:, None, :]
    xk = xk[idx]
    xv = xv[idx]
    pe = _lin(p_r, pr["p1"])
    pe = _relu(_bn(pe, pr["pbn"], (0, 1)))
    pe = _lin(pe, pr["p2"])
    w = xk - xq[:, None, :] + pe
    w = _relu(_bn(w, pr["wbn1"], (0, 1)))
    w = _lin(w, pr["w1"])
    w = _relu(_bn(w, pr["wbn2"], (0, 1)))
    w = _lin(w, pr["w2"])
    w = jax.nn.softmax(w, axis=1)
    v = (xv + pe).reshape(n, nsample, share, out // share)
    return jnp.sum(v * w[:, :, None, :], axis=1).reshape(n, out)


def _pt_block(bp, p, x, idx, nsample, share):
    identity = x
    x = _relu(_bn(_lin(x, bp["l1"]), bp["bn1"], 0))
    x = _relu(_bn(_pt_layer(bp["tr"], p, x, idx, nsample, share),
                  bp["bn2"], 0))
    x = _bn(_lin(x, bp["l3"]), bp["bn3"], 0)
    return _relu(x + identity)


def _transition_down(tp, p, x, stride, nsample):
    if stride == 1:
        return p, _relu(_bn(_lin(x, tp["lin"]), tp["bn"], 0))
    m = p.shape[0] // stride
    sidx = _fps(p, m)
    n_p = p[sidx]
    nidx, _ = _knn(n_p, p, nsample)
    grouped = jnp.concatenate([p[nidx] - n_p[:, None, :], x[nidx]], axis=-1)
    y = _lin(grouped, tp["lin"])
    y = _relu(_bn(y, tp["bn"], (0, 1)))
    return n_p, jnp.max(y, axis=1)


def _tu_head(tp, x):
    g = jnp.mean(x, axis=0, keepdims=True)
    g = _relu(_lin(g, tp["l2"]))
    xc = jnp.concatenate([x, jnp.broadcast_to(g, x.shape)], axis=1)
    return _relu(_bn(_lin(xc, tp["l1"]), tp["l1bn"], 0))


def _tu(tp, p1, x1, p2, x2):
    a = _relu(_bn(_lin(x1, tp["l1"]), tp["l1bn"], 0))
    b = _relu(_bn(_lin(x2, tp["l2"]), tp["l2bn"], 0))
    idx, d2 = _knn(p1, p2, 3)
    w = 1.0 / (jnp.sqrt(jnp.maximum(d2, 1e-12)) + 1e-8)
    w = w / jnp.sum(w, axis=1, keepdims=True)
    return a + jnp.sum(b[idx] * w[:, :, None], axis=1)


def _run_blocks(blocks, p, feats, aux, ns):
    idx, idxf, prf = aux
    n = p.shape[0]
    for bp in blocks:
        if n <= 256:
            feats = _pt_block_fused(bp, feats, prf, idxf, n, ns)
        else:
            feats = _pt_block(bp, p, feats, idx, ns, _SHARE)
    return feats


def _forward(pos, x, params):
    feats = jnp.concatenate([pos, x], axis=1)
    p = pos
    skips = []
    self_aux = []
    for li in range(5):
        ep = params["enc%d" % (li + 1)]
        p, feats = _transition_down(ep["td"], p, feats, _STRIDE[li],
                                    _NSAMPLE[li])
        idx, _ = _knn(p, p, _NSAMPLE[li])
        idxf = idx.reshape(-1, 1)
        prf = (p[idx] - p[:, None, :]).reshape(-1, 3)
        aux = (idx, idxf, prf)
        self_aux.append(aux)
        feats = _run_blocks(ep["blocks"], p, feats, aux, _NSAMPLE[li])
        skips.append((p, feats))
    p5, x5 = skips[4]
    x5 = _tu_head(params["dec5"]["tu"], x5)
    x5 = _run_blocks(params["dec5"]["blocks"], p5, x5, self_aux[4],
                     _NSAMPLE[4])
    cur_p, cur_x = p5, x5
    for name, lv, ns in zip(["dec4", "dec3", "dec2", "dec1"], [3, 2, 1, 0],
                            [_NSAMPLE[3], _NSAMPLE[2], _NSAMPLE[1],
                             _NSAMPLE[0]]):
        p_l, x_l = skips[lv]
        x_new = _tu(params[name]["tu"], p_l, x_l, cur_p, cur_x)
        x_new = _run_blocks(params[name]["blocks"], p_l, x_new,
                            self_aux[lv], ns)
        cur_p, cur_x = p_l, x_new
    out = _lin(cur_x, params["cls"]["l1"])
    out = _relu(_bn(out, params["cls"]["bn"], 0))
    return _lin(out, params["cls"]["l2"])


@jax.jit
def _forward_jit(pos, x, params):
    return _forward(pos, x, params)


def kernel(pos, x, batch, params):
    return _forward_jit(pos, x, params)
